# baseline (speedup 1.0000x reference)
import functools

import jax
import jax.numpy as jnp
from jax import lax
from jax.experimental import pallas as pl
from jax.experimental.pallas import tpu as pltpu
from jax.experimental.pallas import tpu_sc as plsc

NC = 2
NS = 16
M = 2
EB = M * 128
ND = 2
NI = 4
ADT = jnp.bfloat16


def _sc_segsum(state, idx_r, zblk):
    n, d = state.shape
    nb = idx_r.shape[2]
    assert nb % NI == 0 and nb >= NI + 2
    rows_per_sub = -(-(n + 1) // (NS * 16)) * 16
    acc_rows = NS * rows_per_sub
    chunks = []
    r = 0
    while r < rows_per_sub:
        c = min(EB, rows_per_sub - r)
        chunks.append((r, c))
        r += c

    mesh = plsc.VectorSubcoreMesh(core_axis_name="c", subcore_axis_name="s")

    @functools.partial(
        pl.kernel,
        out_type=jax.ShapeDtypeStruct((NC, acc_rows, d), ADT),
        mesh=mesh,
        compiler_params=pltpu.CompilerParams(use_tc_tiling_on_sc=False),
        scratch_types=(
            [pltpu.VMEM((2, M, 128), jnp.int32)] * NI
            + [pltpu.VMEM((EB, d), ADT)] * ND
            + [pltpu.VMEM_SHARED((acc_rows, d), ADT)]
            + [pltpu.SemaphoreType.DMA] * (NI + 2 * ND)
        ),
    )
    def k(state_hbm, idx_hbm, z_hbm, out_hbm, *rest):
        ibufs = rest[:NI]
        bufs = rest[NI:NI + ND]
        acc = rest[NI + ND]
        isems = rest[NI + ND + 1:2 * NI + ND + 1]
        gsems = rest[2 * NI + ND + 1:2 * NI + 2 * ND + 1]
        ssems = rest[2 * NI + 2 * ND + 1:]
        cid = lax.axis_index("c")
        sid = lax.axis_index("s")
        pltpu.sync_copy(z_hbm, bufs[0])
        for r0, c in chunks:
            pltpu.sync_copy(bufs[0].at[pl.ds(0, c)],
                            acc.at[pl.ds(sid * rows_per_sub + r0, c)])
        plsc.subcore_barrier()

        def fire_idx(j, sj):
            q = sj % NI
            pltpu.async_copy(idx_hbm.at[cid, sid, j], ibufs[q], isems[q])

        def wait_idx(sj):
            q = sj % NI
            pltpu.make_async_copy(
                idx_hbm.at[cid, sid, 0], ibufs[q], isems[q]).wait()

        def fire_gathers(sj):
            b, q = sj % ND, sj % NI
            for t in range(M):
                pltpu.async_copy(state_hbm.at[ibufs[q].at[0, t]],
                                 bufs[b].at[pl.ds(t * 128, 128)], gsems[b])

        def wait_gathers(sj):
            b, q = sj % ND, sj % NI
            for t in range(M):
                pltpu.make_async_copy(
                    state_hbm.at[ibufs[q].at[0, t]],
                    bufs[b].at[pl.ds(t * 128, 128)], gsems[b]).wait()

        def fire_scatters(sj):
            b, q = sj % ND, sj % NI
            for t in range(M):
                pltpu.async_copy(bufs[b].at[pl.ds(t * 128, 128)],
                                 acc.at[ibufs[q].at[1, t]], ssems[b],
                                 add=True)

        def wait_scatters(sj):
            b, q = sj % ND, sj % NI
            for t in range(M):
                pltpu.make_async_copy(
                    bufs[b].at[pl.ds(t * 128, 128)],
                    acc.at[ibufs[q].at[1, t]], ssems[b]).wait()

        def step(j, sj, s1=True, s2=True, s3=True, s4=True):
            if s1:
                wait_idx(sj + 1)
            if s2:
                wait_scatters(sj - 1)
            if s3:
                fire_gathers(sj + 1)
            if s4:
                fire_idx(j + 2, sj + 2)
            wait_gathers(sj)
            fire_scatters(sj)

        fire_idx(0, 0)
        fire_idx(1, 1)
        wait_idx(0)
        fire_gathers(0)
        step(0, 0, s2=False)
        step(1, 1)

        def body(i, carry):
            for u in range(NI):
                step(2 + i * NI + u, 2 + u)
            return carry

        lax.fori_loop(0, (nb - 4) // NI, body, 0)
        step(nb - 2, nb - 2, s4=False)
        step(nb - 1, nb - 1, s1=False, s3=False, s4=False)
        wait_scatters(nb - 1)
        plsc.subcore_barrier()
        for t, (r0, c) in enumerate(chunks):
            b = t % ND
            r = sid * rows_per_sub + r0
            pltpu.sync_copy(acc.at[pl.ds(r, c)], bufs[b].at[pl.ds(0, c)])
            pltpu.sync_copy(bufs[b].at[pl.ds(0, c)],
                            out_hbm.at[cid, pl.ds(r, c)])

    return k(state, idx_r, zblk)


def _combine(partials, n):
    d = partials.shape[2]
    rb = 1000

    def body(p_ref, o_ref):
        o_ref[...] = (p_ref[0].astype(jnp.float32)
                      + p_ref[1].astype(jnp.float32))

    return pl.pallas_call(
        body,
        grid=(n // rb,),
        in_specs=[pl.BlockSpec((2, rb, d), lambda i: (0, i, 0))],
        out_specs=pl.BlockSpec((rb, d), lambda i: (i, 0)),
        out_shape=jax.ShapeDtypeStruct((n, d), jnp.float32),
    )(partials)


def kernel(state_input, adj):
    n, d = state_input.shape
    e = adj.shape[1]
    nb = -(-e // (NC * NS * EB))
    nb = max(-(-nb // NI) * NI, NI + 4)
    pad = NC * NS * nb * EB - e
    row_p = jnp.concatenate([adj[0], jnp.zeros((pad,), jnp.int32)])
    col_p = jnp.concatenate([adj[1], jnp.full((pad,), n, jnp.int32)])
    idx_r = jnp.stack(
        [row_p.reshape(NC, NS, nb, M, 128),
         col_p.reshape(NC, NS, nb, M, 128)], axis=3)
    zblk = jnp.zeros((EB, d), ADT)
    partials = _sc_segsum(state_input.astype(ADT), idx_r, zblk)
    out = _combine(partials, n)
    zeros = jnp.zeros_like(state_input)
    return (out, state_input, zeros, out, out)


# speedup vs baseline: 1.9747x; 1.9747x over previous
import functools

import jax
import jax.numpy as jnp
from jax import lax
from jax.experimental import pallas as pl
from jax.experimental.pallas import tpu as pltpu
from jax.experimental.pallas import tpu_sc as plsc

NC = 2
NS = 16
M = 2
EB = M * 128
ND = 2
NI = 4
ADT = jnp.bfloat16


def _sc_segsum(state, idx_r, zblk):
    sp_rows, d = state.shape
    nb = idx_r.shape[2]
    assert nb % NI == 0 and nb >= NI + 2
    n = sp_rows
    rows_per_sub = -(-(n + 1) // (NS * 16)) * 16
    acc_rows = NS * rows_per_sub
    chunks = []
    r = 0
    while r < rows_per_sub:
        c = min(EB, rows_per_sub - r)
        chunks.append((r, c))
        r += c

    mesh = plsc.VectorSubcoreMesh(core_axis_name="c", subcore_axis_name="s")

    @functools.partial(
        pl.kernel,
        out_type=jax.ShapeDtypeStruct((NC, acc_rows, d), ADT),
        mesh=mesh,
        compiler_params=pltpu.CompilerParams(use_tc_tiling_on_sc=False),
        scratch_types=(
            [pltpu.VMEM((2, M, 128), jnp.int32)] * NI
            + [pltpu.VMEM((EB, d), ADT)] * ND
            + [pltpu.VMEM_SHARED((acc_rows, d), ADT)]
            + [pltpu.VMEM_SHARED((sp_rows, d), ADT)]
            + [pltpu.SemaphoreType.DMA] * (NI + 2 * ND)
        ),
    )
    def k(state_hbm, idx_hbm, z_hbm, out_hbm, *rest):
        ibufs = rest[:NI]
        bufs = rest[NI:NI + ND]
        acc = rest[NI + ND]
        state_sp = rest[NI + ND + 1]
        isems = rest[NI + ND + 2:2 * NI + ND + 2]
        gsems = rest[2 * NI + ND + 2:2 * NI + 2 * ND + 2]
        ssems = rest[2 * NI + 2 * ND + 2:]
        cid = lax.axis_index("c")
        sid = lax.axis_index("s")
        spt = sp_rows // NS
        pltpu.sync_copy(state_hbm.at[pl.ds(sid * spt, spt)],
                        state_sp.at[pl.ds(sid * spt, spt)])
        pltpu.sync_copy(z_hbm, bufs[0])
        for r0, c in chunks:
            pltpu.sync_copy(bufs[0].at[pl.ds(0, c)],
                            acc.at[pl.ds(sid * rows_per_sub + r0, c)])
        plsc.subcore_barrier()

        def fire_idx(j, sj):
            q = sj % NI
            pltpu.async_copy(idx_hbm.at[cid, sid, j], ibufs[q], isems[q])

        def wait_idx(sj):
            q = sj % NI
            pltpu.make_async_copy(
                idx_hbm.at[cid, sid, 0], ibufs[q], isems[q]).wait()

        def fire_gathers(sj):
            b, q = sj % ND, sj % NI
            for t in range(M):
                pltpu.async_copy(state_sp.at[ibufs[q].at[0, t]],
                                 bufs[b].at[pl.ds(t * 128, 128)], gsems[b])

        def wait_gathers(sj):
            b, q = sj % ND, sj % NI
            for t in range(M):
                pltpu.make_async_copy(
                    state_sp.at[ibufs[q].at[0, t]],
                    bufs[b].at[pl.ds(t * 128, 128)], gsems[b]).wait()

        def fire_scatters(sj):
            b, q = sj % ND, sj % NI
            for t in range(M):
                pltpu.async_copy(bufs[b].at[pl.ds(t * 128, 128)],
                                 acc.at[ibufs[q].at[1, t]], ssems[b],
                                 add=True)

        def wait_scatters(sj):
            b, q = sj % ND, sj % NI
            for t in range(M):
                pltpu.make_async_copy(
                    bufs[b].at[pl.ds(t * 128, 128)],
                    acc.at[ibufs[q].at[1, t]], ssems[b]).wait()

        def step(j, sj, s1=True, s2=True, s3=True, s4=True):
            if s1:
                wait_idx(sj + 1)
            if s2:
                wait_scatters(sj - 1)
            if s3:
                fire_gathers(sj + 1)
            if s4:
                fire_idx(j + 2, sj + 2)
            wait_gathers(sj)
            fire_scatters(sj)

        fire_idx(0, 0)
        fire_idx(1, 1)
        wait_idx(0)
        fire_gathers(0)
        step(0, 0, s2=False)
        step(1, 1)

        def body(i, carry):
            for u in range(NI):
                step(2 + i * NI + u, 2 + u)
            return carry

        lax.fori_loop(0, (nb - 4) // NI, body, 0)
        step(nb - 2, nb - 2, s4=False)
        step(nb - 1, nb - 1, s1=False, s3=False, s4=False)
        wait_scatters(nb - 1)
        plsc.subcore_barrier()
        for t, (r0, c) in enumerate(chunks):
            b = t % ND
            r = sid * rows_per_sub + r0
            pltpu.sync_copy(acc.at[pl.ds(r, c)], bufs[b].at[pl.ds(0, c)])
            pltpu.sync_copy(bufs[b].at[pl.ds(0, c)],
                            out_hbm.at[cid, pl.ds(r, c)])

    return k(state, idx_r, zblk)


def _combine(partials, n):
    d = partials.shape[2]
    rb = 1000

    def body(p_ref, o_ref):
        o_ref[...] = (p_ref[0].astype(jnp.float32)
                      + p_ref[1].astype(jnp.float32))

    return pl.pallas_call(
        body,
        grid=(n // rb,),
        in_specs=[pl.BlockSpec((2, rb, d), lambda i: (0, i, 0))],
        out_specs=pl.BlockSpec((rb, d), lambda i: (i, 0)),
        out_shape=jax.ShapeDtypeStruct((n, d), jnp.float32),
    )(partials)


def kernel(state_input, adj):
    n, d = state_input.shape
    e = adj.shape[1]
    nb = -(-e // (NC * NS * EB))
    nb = max(-(-nb // NI) * NI, NI + 4)
    pad = NC * NS * nb * EB - e
    row_p = jnp.concatenate([adj[0], jnp.zeros((pad,), jnp.int32)])
    col_p = jnp.concatenate([adj[1], jnp.full((pad,), n, jnp.int32)])
    idx_r = jnp.stack(
        [row_p.reshape(NC, NS, nb, M, 128),
         col_p.reshape(NC, NS, nb, M, 128)], axis=3)
    zblk = jnp.zeros((EB, d), ADT)
    sp_rows = -(-n // (NS * 16)) * NS * 16
    state_p = jnp.pad(state_input.astype(ADT), ((0, sp_rows - n), (0, 0)))
    partials = _sc_segsum(state_p, idx_r, zblk)
    out = _combine(partials, n)
    zeros = jnp.zeros_like(state_input)
    return (out, state_input, zeros, out, out)
